# Initial kernel scaffold; baseline (speedup 1.0000x reference)
import jax
import jax.numpy as jnp
from jax.experimental import pallas as pl


def kernel(inputs, codebook):
    raise NotImplementedError("write your pallas kernel here")


# speedup vs baseline: 1.5605x; 1.5605x over previous
import functools

import jax
import jax.numpy as jnp
from jax import lax
from jax.experimental import pallas as pl
from jax.experimental.pallas import tpu as pltpu
from jax.experimental.pallas import tpu_sc as plsc


def _argmin_body(x_ref, l2in_ref, c_ref, l2c_ref, idx_ref, minsum_ref, ct_ref):
    i = pl.program_id(0)
    ct_blk = ct_ref.shape[0]
    x = x_ref[...]
    c = c_ref[...]
    dot = jnp.dot(x, c, preferred_element_type=jnp.float32)
    dist = l2in_ref[...] - 2.0 * dot + l2c_ref[...]
    minval = jnp.min(dist, axis=1, keepdims=True)
    iota = lax.broadcasted_iota(jnp.int32, dist.shape, 1)
    big = jnp.int32(2**30)
    idx = jnp.min(jnp.where(dist == minval, iota, big), axis=1, keepdims=True)
    idx_ref[...] = idx
    s = jnp.sum(minval).reshape(1, 1)

    @pl.when(i == 0)
    def _():
        minsum_ref[...] = s

    @pl.when(i != 0)
    def _():
        minsum_ref[...] = minsum_ref[...] + s

    ct_ref[...] = c_ref[:, pl.ds(i * ct_blk, ct_blk)].T


def _argmin_stage(flatten, l2_in, codebook, l2_c, block_rows):
    n, d = flatten.shape
    e = codebook.shape[1]
    grid = n // block_rows
    ct_blk = e // grid
    return pl.pallas_call(
        _argmin_body,
        grid=(grid,),
        in_specs=[
            pl.BlockSpec((block_rows, d), lambda i: (i, 0)),
            pl.BlockSpec((block_rows, 1), lambda i: (i, 0)),
            pl.BlockSpec((d, e), lambda i: (0, 0)),
            pl.BlockSpec((1, e), lambda i: (0, 0)),
        ],
        out_specs=[
            pl.BlockSpec((block_rows, 1), lambda i: (i, 0)),
            pl.BlockSpec((1, 1), lambda i: (0, 0)),
            pl.BlockSpec((ct_blk, d), lambda i: (i, 0)),
        ],
        out_shape=[
            jax.ShapeDtypeStruct((n, 1), jnp.int32),
            jax.ShapeDtypeStruct((1, 1), jnp.float32),
            jax.ShapeDtypeStruct((e, d), jnp.float32),
        ],
    )(flatten, l2_in, codebook, l2_c)


_NUM_WORKERS = 32
_GATHER_CHUNK = 128


def _make_gather(n, d):
    b_per_w = n // _NUM_WORKERS
    n_chunks = b_per_w // _GATHER_CHUNK
    mesh = plsc.VectorSubcoreMesh(core_axis_name="c", subcore_axis_name="s")

    @functools.partial(
        pl.kernel,
        mesh=mesh,
        out_type=jax.ShapeDtypeStruct((n, d), jnp.float32),
        scratch_types=[
            pltpu.VMEM((_GATHER_CHUNK,), jnp.int32),
            pltpu.VMEM((_GATHER_CHUNK, d), jnp.float32),
            pltpu.SemaphoreType.DMA,
        ],
    )
    def _gather(table_hbm, idx_hbm, out_hbm, idx_v, rows_v, sem):
        wid = lax.axis_index("s") * 2 + lax.axis_index("c")
        base = wid * b_per_w
        for k in range(n_chunks):
            off = base + k * _GATHER_CHUNK
            pltpu.sync_copy(idx_hbm.at[pl.ds(off, _GATHER_CHUNK)], idx_v)
            pltpu.async_copy(table_hbm.at[idx_v], rows_v, sem).wait()
            pltpu.sync_copy(rows_v, out_hbm.at[pl.ds(off, _GATHER_CHUNK)])

    return _gather


def kernel(inputs, codebook):
    d = codebook.shape[0]
    flatten = inputs.reshape(-1, d)
    n = flatten.shape[0]
    l2_in = jnp.sum(flatten ** 2, axis=1, keepdims=True)
    l2_c = jnp.sum(codebook ** 2, axis=0, keepdims=True)

    idx2d, minsum, c_t = _argmin_stage(flatten, l2_in, codebook, l2_c, 512)
    idx = idx2d.reshape(-1)

    quantized = _make_gather(n, d)(c_t, idx)

    embedding_loss = minsum[0, 0] / jnp.float32(n * d)
    return (quantized.reshape(inputs.shape), embedding_loss)
